# Initial kernel scaffold; baseline (speedup 1.0000x reference)
import jax
import jax.numpy as jnp
from jax.experimental import pallas as pl


def kernel(x, y, deg_g, deg_lg, edge_index_g, edge_index_lg, pm_pd, Wt, bt, Wg, bg):
    raise NotImplementedError("write your pallas kernel here")


# speedup vs baseline: 1.6431x; 1.6431x over previous
import functools

import jax
import jax.numpy as jnp
from jax import lax
from jax.experimental import pallas as pl
from jax.experimental.pallas import tpu as pltpu
from jax.experimental.pallas import tpu_sc as plsc

D = 128
NC = 2
NS = 16
F32 = jnp.float32
I32 = jnp.int32


def _mm5_body(x_ref, w_ref, deg_ref, o0, o1, o2, o3, o4):
    x = x_ref[...]
    p = lax.dot_general(x, w_ref[...], (((1,), (1,)), ((), ())),
                        preferred_element_type=F32)
    o0[...] = p[:, 0 * D:1 * D]
    o1[...] = p[:, 1 * D:2 * D] * deg_ref[...]
    o2[...] = p[:, 2 * D:3 * D]
    o3[...] = p[:, 3 * D:4 * D]
    o4[...] = p[:, 4 * D:5 * D]


def _mm5(x, w, deg, bm):
    m = x.shape[0]
    blk = lambda i: (i, 0)
    out = jax.ShapeDtypeStruct((m, D), F32)
    return pl.pallas_call(
        _mm5_body,
        grid=(m // bm,),
        in_specs=[
            pl.BlockSpec((bm, D), blk),
            pl.BlockSpec((5 * D, D), lambda i: (0, 0)),
            pl.BlockSpec((bm, 1), blk),
        ],
        out_specs=[pl.BlockSpec((bm, D), blk)] * 5,
        out_shape=[out] * 5,
    )(x, w, deg)


def _add_body(a_ref, b_ref, o_ref):
    o_ref[...] = a_ref[...] + b_ref[...]


def _add(a, b, bm):
    m = a.shape[0]
    blk = lambda i: (i, 0)
    return pl.pallas_call(
        _add_body,
        grid=(m // bm,),
        in_specs=[pl.BlockSpec((bm, D), blk)] * 2,
        out_specs=pl.BlockSpec((bm, D), blk),
        out_shape=jax.ShapeDtypeStruct((m, D), F32),
    )(a, b)


def _asm_body(p0_ref, p1_ref, agg_ref, pm_ref, b_ref, o_ref):
    pre = p0_ref[...] + p1_ref[...] + agg_ref[...] + pm_ref[...] + b_ref[...]
    col = lax.broadcasted_iota(I32, pre.shape, 1)
    o_ref[...] = jnp.where((col >= D // 2) & (pre < 0.0), 0.0, pre)


def _assemble(p0, p1, agg, pm, bias, bm):
    m = p0.shape[0]
    blk = lambda i: (i, 0)
    return pl.pallas_call(
        _asm_body,
        grid=(m // bm,),
        in_specs=[pl.BlockSpec((bm, D), blk)] * 4 +
                 [pl.BlockSpec((1, D), lambda i: (0, 0))],
        out_specs=pl.BlockSpec((bm, D), blk),
        out_shape=jax.ShapeDtypeStruct((m, D), F32),
    )(p0, p1, agg, pm, bias)


_W = 2000


def _segsum_body(ne, rng, rounds, chunk, nsrcbits,
                 table, src_hbm, dst_hbm, zeros_hbm, out_hbm,
                 srcw, dstw, packb, srcidx, dstidx, rows, acc, sem):
    c = lax.axis_index("c")
    s = lax.axis_index("s")
    ept = ne // NS
    accr = acc.shape[0]
    zrows = accr // NS
    orows = rng // NS
    srcmask = (1 << nsrcbits) - 1
    lane = lax.iota(I32, 16)
    pad_vec = lane | ((rng + lane) << nsrcbits)
    trash = packb.shape[0] - 1

    def flush(koff, _):
        def unpack_step(k, _):
            v = packb[pl.ds(koff * chunk + k * 16, 16)]
            srcidx[pl.ds(k * 16, 16)] = v & srcmask
            dstidx[pl.ds(k * 16, 16)] = lax.shift_right_logical(v, nsrcbits)
            return 0
        lax.fori_loop(0, chunk // 16, unpack_step, 0)
        pltpu.async_copy(table.at[srcidx], rows, sem).wait()
        pltpu.sync_copy(rows, acc.at[dstidx], add=True)
        return 0

    for r in range(rounds):
        seg_lo = (r * NC + c) * rng

        def zero_step(j, _):
            pltpu.sync_copy(zeros_hbm, acc.at[pl.ds(s * zrows + j * 64, 64)])
            return 0
        lax.fori_loop(0, zrows // 64, zero_step, 0)
        if zrows % 64:
            pltpu.sync_copy(
                zeros_hbm.at[pl.ds(0, zrows % 64)],
                acc.at[pl.ds(s * zrows + (zrows // 64) * 64, zrows % 64)])
        plsc.subcore_barrier()

        def window_step(w, m):
            pltpu.sync_copy(src_hbm.at[pl.ds(s * ept + w * _W, _W)], srcw)
            pltpu.sync_copy(dst_hbm.at[pl.ds(s * ept + w * _W, _W)], dstw)

            def filt_step(i, m):
                vd = dstw[pl.ds(i * 16, 16)]
                vs = srcw[pl.ds(i * 16, 16)]
                msk = (vd >= seg_lo) & (vd < seg_lo + rng)
                packed = vs | ((vd - seg_lo) << nsrcbits)
                rank = plsc.cumsum(jnp.where(msk, 1, 0))
                idx = jnp.where(msk, m + rank - 1, trash)
                plsc.store_scatter(packb, [idx], packed)
                return m + jnp.max(rank)
            m = lax.fori_loop(0, _W // 16, filt_step, m)

            nfull = m // chunk
            lax.fori_loop(0, nfull, flush, 0)
            tail = m - nfull * chunk

            def move_step(k, _):
                packb[pl.ds(k * 16, 16)] = packb[pl.ds(nfull * chunk + k * 16, 16)]
                return 0
            lax.fori_loop(0, jnp.where(nfull > 0, (tail + 15) // 16, 0),
                          move_step, 0)
            return tail
        m = lax.fori_loop(0, ept // _W, window_step, jnp.int32(0))

        mr = ((m + chunk - 1) // chunk) * chunk

        def pad_step(j, _):
            packb[pl.ds(m + j * 16, 16)] = pad_vec
            return 0
        lax.fori_loop(0, (mr - m + 15) // 16, pad_step, 0)
        lax.fori_loop(0, mr // chunk, flush, 0)
        plsc.subcore_barrier()

        pltpu.sync_copy(acc.at[pl.ds(s * orows, orows)],
                        out_hbm.at[pl.ds(seg_lo + s * orows, orows)])
        plsc.subcore_barrier()


def _segsum(table, src, dst, rng, rounds, chunk):
    ne = src.shape[0]
    nseg = rng * NC * rounds
    nsrcbits = 18
    accr = rng + 128
    assert rng % 128 == 0 and chunk % 16 == 0 and (ne // NS) % _W == 0
    zeros = jnp.zeros((64, D), F32)
    mesh = plsc.VectorSubcoreMesh(core_axis_name="c", subcore_axis_name="s")
    body = functools.partial(_segsum_body, ne, rng, rounds, chunk, nsrcbits)
    k = pl.kernel(
        body,
        out_type=jax.ShapeDtypeStruct((nseg, D), F32),
        mesh=mesh,
        scratch_types=[
            pltpu.VMEM((_W,), I32),
            pltpu.VMEM((_W,), I32),
            pltpu.VMEM((_W + chunk + 32,), I32),
            pltpu.VMEM((chunk,), I32),
            pltpu.VMEM((chunk,), I32),
            pltpu.VMEM((chunk, D), F32),
            pltpu.VMEM_SHARED((accr, D), F32),
            pltpu.SemaphoreType.DMA,
        ],
        compiler_params=pltpu.CompilerParams(needs_layout_passes=False),
    )
    return k(table, src, dst, zeros)


def _gather_body(ni, chunk, table, idx_hbm, out_hbm, idxv, rows, sem):
    c = lax.axis_index("c")
    s = lax.axis_index("s")
    wid = s * NC + c
    ipt = ni // (NC * NS)
    pltpu.sync_copy(idx_hbm.at[pl.ds(wid * ipt, ipt)], idxv)

    def chunk_step(j, _):
        pltpu.async_copy(table.at[idxv.at[pl.ds(j * chunk, chunk)]],
                         rows, sem).wait()
        pltpu.sync_copy(rows, out_hbm.at[pl.ds(wid * ipt + j * chunk, chunk)])
        return 0
    lax.fori_loop(0, ipt // chunk, chunk_step, 0)


def _gather(table, idx, chunk):
    ni = idx.shape[0]
    mesh = plsc.VectorSubcoreMesh(core_axis_name="c", subcore_axis_name="s")
    body = functools.partial(_gather_body, ni, chunk)
    k = pl.kernel(
        body,
        out_type=jax.ShapeDtypeStruct((ni, D), F32),
        mesh=mesh,
        scratch_types=[
            pltpu.VMEM((ni // (NC * NS),), I32),
            pltpu.VMEM((chunk, D), F32),
            pltpu.SemaphoreType.DMA,
        ],
        compiler_params=pltpu.CompilerParams(needs_layout_passes=False),
    )
    return k(table, idx)


def kernel(x, y, deg_g, deg_lg, edge_index_g, edge_index_lg, pm_pd, Wt, bt, Wg, bg):
    n_nodes, _ = x.shape
    n_edges, _ = y.shape

    wx = jnp.concatenate([Wt[0], Wt[1], Wt[3], Wt[4], Wg[2]], axis=0)
    wy = jnp.concatenate([Wg[0], Wg[1], Wg[3], Wg[4], Wt[2]], axis=0)
    bx = jnp.sum(bt, axis=0).reshape(1, D)
    by = jnp.sum(bg, axis=0).reshape(1, D)
    src_g, dst_g = edge_index_g[0], edge_index_g[1]
    src_l, dst_l = edge_index_lg[0], edge_index_lg[1]
    eids = jnp.arange(n_edges, dtype=I32)

    x0, x1, x2, x3, x4 = _mm5(x, wx, deg_g, 1000)
    y0, y1, y2, y3, y4 = _mm5(y, wy, deg_lg, 1000)

    a1 = _segsum(x3, src_g, dst_g, 5120, 1, 256)[:n_nodes]
    y4t, _ = lax.optimization_barrier((y4, a1))
    py = _segsum(y4t, eids, dst_g, 5120, 1, 256)[:n_nodes]
    y3t, _ = lax.optimization_barrier((y3, py))
    b1 = _segsum(y3t, src_l, dst_l, 10624, 8, 256)[:n_edges]

    tg = _add(x2, a1, 1000)
    tl = _add(y2, b1, 1000)

    tgt, _ = lax.optimization_barrier((tg, b1))
    a2 = _segsum(tgt, src_g, dst_g, 5120, 1, 256)[:n_nodes]
    tlt, _ = lax.optimization_barrier((tl, a2))
    b2 = _segsum(tlt, src_l, dst_l, 10624, 8, 256)[:n_edges]

    x4t, _ = lax.optimization_barrier((x4, b2))
    px = _gather(x4t, pm_pd.astype(I32), 200)

    x_new = _assemble(x0, x1, a2, py, bx, 1000)
    y_new = _assemble(y0, y1, b2, px, by, 1000)
    return (x_new, y_new)


# speedup vs baseline: 1.7297x; 1.0527x over previous
import functools

import jax
import jax.numpy as jnp
from jax import lax
from jax.experimental import pallas as pl
from jax.experimental.pallas import tpu as pltpu
from jax.experimental.pallas import tpu_sc as plsc

D = 128
NC = 2
NS = 16
F32 = jnp.float32
I32 = jnp.int32


def _mm5_body(x_ref, w_ref, deg_ref, o0, o1, o2, o3, o4):
    x = x_ref[...]
    p = lax.dot_general(x, w_ref[...], (((1,), (1,)), ((), ())),
                        preferred_element_type=F32)
    o0[...] = p[:, 0 * D:1 * D]
    o1[...] = p[:, 1 * D:2 * D] * deg_ref[...]
    o2[...] = p[:, 2 * D:3 * D]
    o3[...] = p[:, 3 * D:4 * D]
    o4[...] = p[:, 4 * D:5 * D]


def _mm5(x, w, deg, bm):
    m = x.shape[0]
    blk = lambda i: (i, 0)
    out = jax.ShapeDtypeStruct((m, D), F32)
    return pl.pallas_call(
        _mm5_body,
        grid=(m // bm,),
        in_specs=[
            pl.BlockSpec((bm, D), blk),
            pl.BlockSpec((5 * D, D), lambda i: (0, 0)),
            pl.BlockSpec((bm, 1), blk),
        ],
        out_specs=[pl.BlockSpec((bm, D), blk)] * 5,
        out_shape=[out] * 5,
    )(x, w, deg)


def _add_body(a_ref, b_ref, o_ref):
    o_ref[...] = a_ref[...] + b_ref[...]


def _add(a, b, bm):
    m = a.shape[0]
    blk = lambda i: (i, 0)
    return pl.pallas_call(
        _add_body,
        grid=(m // bm,),
        in_specs=[pl.BlockSpec((bm, D), blk)] * 2,
        out_specs=pl.BlockSpec((bm, D), blk),
        out_shape=jax.ShapeDtypeStruct((m, D), F32),
    )(a, b)


def _asm_body(p0_ref, p1_ref, agg_ref, pm_ref, b_ref, o_ref):
    pre = p0_ref[...] + p1_ref[...] + agg_ref[...] + pm_ref[...] + b_ref[...]
    col = lax.broadcasted_iota(I32, pre.shape, 1)
    o_ref[...] = jnp.where((col >= D // 2) & (pre < 0.0), 0.0, pre)


def _assemble(p0, p1, agg, pm, bias, bm):
    m = p0.shape[0]
    blk = lambda i: (i, 0)
    return pl.pallas_call(
        _asm_body,
        grid=(m // bm,),
        in_specs=[pl.BlockSpec((bm, D), blk)] * 4 +
                 [pl.BlockSpec((1, D), lambda i: (0, 0))],
        out_specs=pl.BlockSpec((bm, D), blk),
        out_shape=jax.ShapeDtypeStruct((m, D), F32),
    )(p0, p1, agg, pm, bias)


_W = 2000


def _segsum_body(ne, rng, rounds, chunk, nsrcbits,
                 table, src_hbm, dst_hbm, zeros_hbm, out_hbm,
                 srcw, dstw, packb,
                 srcidx0, dstidx0, rows0, srcidx1, dstidx1, rows1,
                 acc, sem0, sem1):
    c = lax.axis_index("c")
    s = lax.axis_index("s")
    ept = ne // NS
    accr = acc.shape[0]
    zrows = accr // NS
    orows = rng // NS
    srcmask = (1 << nsrcbits) - 1
    lane = lax.iota(I32, 16)
    pad_vec = lane | ((rng + lane) << nsrcbits)
    trash = packb.shape[0] - 1
    bufs = ((srcidx0, dstidx0, rows0, sem0), (srcidx1, dstidx1, rows1, sem1))

    def start(koff, b):
        srcidx, dstidx, rows, sem = bufs[b]

        def unpack_step(k, _):
            v = packb[pl.ds(koff * chunk + k * 16, 16)]
            srcidx[pl.ds(k * 16, 16)] = v & srcmask
            dstidx[pl.ds(k * 16, 16)] = lax.shift_right_logical(v, nsrcbits)
            return 0
        lax.fori_loop(0, chunk // 16, unpack_step, 0)
        pltpu.async_copy(table.at[srcidx], rows, sem)

    def finish(b):
        srcidx, dstidx, rows, sem = bufs[b]
        pltpu.make_async_copy(table.at[srcidx], rows, sem).wait()
        pltpu.sync_copy(rows, acc.at[dstidx], add=True)

    def flush_many(nch, base):
        @pl.when(nch > 0)
        def _():
            start(base, 0)

            def pipe(k, _):
                @pl.when(k % 2 == 0)
                def _():
                    start(base + k + 1, 1)
                    finish(0)

                @pl.when(k % 2 == 1)
                def _():
                    start(base + k + 1, 0)
                    finish(1)
                return 0
            lax.fori_loop(0, nch - 1, pipe, 0)
            @pl.when((nch - 1) % 2 == 0)
            def _():
                finish(0)
            @pl.when((nch - 1) % 2 == 1)
            def _():
                finish(1)

    for r in range(rounds):
        seg_lo = (r * NC + c) * rng

        def zero_step(j, _):
            pltpu.sync_copy(zeros_hbm, acc.at[pl.ds(s * zrows + j * 64, 64)])
            return 0
        lax.fori_loop(0, zrows // 64, zero_step, 0)
        if zrows % 64:
            pltpu.sync_copy(
                zeros_hbm.at[pl.ds(0, zrows % 64)],
                acc.at[pl.ds(s * zrows + (zrows // 64) * 64, zrows % 64)])
        plsc.subcore_barrier()

        def window_step(w, m):
            pltpu.sync_copy(src_hbm.at[pl.ds(s * ept + w * _W, _W)], srcw)
            pltpu.sync_copy(dst_hbm.at[pl.ds(s * ept + w * _W, _W)], dstw)

            def filt_step(i, m):
                vd = dstw[pl.ds(i * 16, 16)]
                vs = srcw[pl.ds(i * 16, 16)]
                msk = (vd >= seg_lo) & (vd < seg_lo + rng)
                packed = vs | ((vd - seg_lo) << nsrcbits)
                rank = plsc.cumsum(jnp.where(msk, 1, 0))
                idx = jnp.where(msk, m + rank - 1, trash)
                plsc.store_scatter(packb, [idx], packed)
                return m + jnp.max(rank)
            m = lax.fori_loop(0, _W // 16, filt_step, m)

            nfull = m // chunk
            flush_many(nfull, 0)
            tail = m - nfull * chunk

            def move_step(k, _):
                packb[pl.ds(k * 16, 16)] = packb[pl.ds(nfull * chunk + k * 16, 16)]
                return 0
            lax.fori_loop(0, jnp.where(nfull > 0, (tail + 15) // 16, 0),
                          move_step, 0)
            return tail
        m = lax.fori_loop(0, ept // _W, window_step, jnp.int32(0))

        mr = ((m + chunk - 1) // chunk) * chunk

        def pad_step(j, _):
            packb[pl.ds(m + j * 16, 16)] = pad_vec
            return 0
        lax.fori_loop(0, (mr - m + 15) // 16, pad_step, 0)
        flush_many(mr // chunk, 0)
        plsc.subcore_barrier()

        pltpu.sync_copy(acc.at[pl.ds(s * orows, orows)],
                        out_hbm.at[pl.ds(seg_lo + s * orows, orows)])
        plsc.subcore_barrier()


def _segsum(table, src, dst, rng, rounds, chunk):
    ne = src.shape[0]
    nseg = rng * NC * rounds
    nsrcbits = 18
    accr = rng + 128
    assert rng % 128 == 0 and chunk % 16 == 0 and (ne // NS) % _W == 0
    zeros = jnp.zeros((64, D), F32)
    mesh = plsc.VectorSubcoreMesh(core_axis_name="c", subcore_axis_name="s")
    body = functools.partial(_segsum_body, ne, rng, rounds, chunk, nsrcbits)
    k = pl.kernel(
        body,
        out_type=jax.ShapeDtypeStruct((nseg, D), F32),
        mesh=mesh,
        scratch_types=[
            pltpu.VMEM((_W,), I32),
            pltpu.VMEM((_W,), I32),
            pltpu.VMEM((_W + chunk + 32,), I32),
            pltpu.VMEM((chunk,), I32),
            pltpu.VMEM((chunk,), I32),
            pltpu.VMEM((chunk, D), F32),
            pltpu.VMEM((chunk,), I32),
            pltpu.VMEM((chunk,), I32),
            pltpu.VMEM((chunk, D), F32),
            pltpu.VMEM_SHARED((accr, D), F32),
            pltpu.SemaphoreType.DMA,
            pltpu.SemaphoreType.DMA,
        ],
        compiler_params=pltpu.CompilerParams(needs_layout_passes=False),
    )
    return k(table, src, dst, zeros)


def _gather_body(ni, chunk, table, idx_hbm, out_hbm, idxv, rows, sem):
    c = lax.axis_index("c")
    s = lax.axis_index("s")
    wid = s * NC + c
    ipt = ni // (NC * NS)
    pltpu.sync_copy(idx_hbm.at[pl.ds(wid * ipt, ipt)], idxv)

    def chunk_step(j, _):
        pltpu.async_copy(table.at[idxv.at[pl.ds(j * chunk, chunk)]],
                         rows, sem).wait()
        pltpu.sync_copy(rows, out_hbm.at[pl.ds(wid * ipt + j * chunk, chunk)])
        return 0
    lax.fori_loop(0, ipt // chunk, chunk_step, 0)


def _gather(table, idx, chunk):
    ni = idx.shape[0]
    mesh = plsc.VectorSubcoreMesh(core_axis_name="c", subcore_axis_name="s")
    body = functools.partial(_gather_body, ni, chunk)
    k = pl.kernel(
        body,
        out_type=jax.ShapeDtypeStruct((ni, D), F32),
        mesh=mesh,
        scratch_types=[
            pltpu.VMEM((ni // (NC * NS),), I32),
            pltpu.VMEM((chunk, D), F32),
            pltpu.SemaphoreType.DMA,
        ],
        compiler_params=pltpu.CompilerParams(needs_layout_passes=False),
    )
    return k(table, idx)


def kernel(x, y, deg_g, deg_lg, edge_index_g, edge_index_lg, pm_pd, Wt, bt, Wg, bg):
    n_nodes, _ = x.shape
    n_edges, _ = y.shape

    wx = jnp.concatenate([Wt[0], Wt[1], Wt[3], Wt[4], Wg[2]], axis=0)
    wy = jnp.concatenate([Wg[0], Wg[1], Wg[3], Wg[4], Wt[2]], axis=0)
    bx = jnp.sum(bt, axis=0).reshape(1, D)
    by = jnp.sum(bg, axis=0).reshape(1, D)
    src_g, dst_g = edge_index_g[0], edge_index_g[1]
    src_l, dst_l = edge_index_lg[0], edge_index_lg[1]
    eids = jnp.arange(n_edges, dtype=I32)

    x0, x1, x2, x3, x4 = _mm5(x, wx, deg_g, 1000)
    y0, y1, y2, y3, y4 = _mm5(y, wy, deg_lg, 1000)

    a1 = _segsum(x3, src_g, dst_g, 5120, 1, 128)[:n_nodes]
    y4t, _ = lax.optimization_barrier((y4, a1))
    py = _segsum(y4t, eids, dst_g, 5120, 1, 128)[:n_nodes]
    y3t, _ = lax.optimization_barrier((y3, py))
    b1 = _segsum(y3t, src_l, dst_l, 10624, 8, 128)[:n_edges]

    tg = _add(x2, a1, 1000)
    tl = _add(y2, b1, 1000)

    tgt, _ = lax.optimization_barrier((tg, b1))
    a2 = _segsum(tgt, src_g, dst_g, 5120, 1, 128)[:n_nodes]
    tlt, _ = lax.optimization_barrier((tl, a2))
    b2 = _segsum(tlt, src_l, dst_l, 10624, 8, 128)[:n_edges]

    x4t, _ = lax.optimization_barrier((x4, b2))
    px = _gather(x4t, pm_pd.astype(I32), 200)

    x_new = _assemble(x0, x1, a2, py, bx, 1000)
    y_new = _assemble(y0, y1, b2, px, by, 1000)
    return (x_new, y_new)


# speedup vs baseline: 1.7412x; 1.0067x over previous
import functools

import jax
import jax.numpy as jnp
from jax import lax
from jax.experimental import pallas as pl
from jax.experimental.pallas import tpu as pltpu
from jax.experimental.pallas import tpu_sc as plsc

D = 128
NC = 2
NS = 16
F32 = jnp.float32
I32 = jnp.int32


def _mm5_body(x_ref, w_ref, deg_ref, o0, o1, o2, o3, o4):
    x = x_ref[...]
    p = lax.dot_general(x, w_ref[...], (((1,), (1,)), ((), ())),
                        preferred_element_type=F32)
    o0[...] = p[:, 0 * D:1 * D]
    o1[...] = p[:, 1 * D:2 * D] * deg_ref[...]
    o2[...] = p[:, 2 * D:3 * D]
    o3[...] = p[:, 3 * D:4 * D]
    o4[...] = p[:, 4 * D:5 * D]


def _mm5(x, w, deg, bm):
    m = x.shape[0]
    blk = lambda i: (i, 0)
    out = jax.ShapeDtypeStruct((m, D), F32)
    return pl.pallas_call(
        _mm5_body,
        grid=(m // bm,),
        in_specs=[
            pl.BlockSpec((bm, D), blk),
            pl.BlockSpec((5 * D, D), lambda i: (0, 0)),
            pl.BlockSpec((bm, 1), blk),
        ],
        out_specs=[pl.BlockSpec((bm, D), blk)] * 5,
        out_shape=[out] * 5,
    )(x, w, deg)


def _add_body(a_ref, b_ref, o_ref):
    o_ref[...] = a_ref[...] + b_ref[...]


def _add(a, b, bm):
    m = a.shape[0]
    blk = lambda i: (i, 0)
    return pl.pallas_call(
        _add_body,
        grid=(m // bm,),
        in_specs=[pl.BlockSpec((bm, D), blk)] * 2,
        out_specs=pl.BlockSpec((bm, D), blk),
        out_shape=jax.ShapeDtypeStruct((m, D), F32),
    )(a, b)


def _asm_body(p0_ref, p1_ref, agg_ref, pm_ref, b_ref, o_ref):
    pre = p0_ref[...] + p1_ref[...] + agg_ref[...] + pm_ref[...] + b_ref[...]
    col = lax.broadcasted_iota(I32, pre.shape, 1)
    o_ref[...] = jnp.where((col >= D // 2) & (pre < 0.0), 0.0, pre)


def _assemble(p0, p1, agg, pm, bias, bm):
    m = p0.shape[0]
    blk = lambda i: (i, 0)
    return pl.pallas_call(
        _asm_body,
        grid=(m // bm,),
        in_specs=[pl.BlockSpec((bm, D), blk)] * 4 +
                 [pl.BlockSpec((1, D), lambda i: (0, 0))],
        out_specs=pl.BlockSpec((bm, D), blk),
        out_shape=jax.ShapeDtypeStruct((m, D), F32),
    )(p0, p1, agg, pm, bias)


_W = 2000


def _segsum_body(ne, rng, rounds, chunk, nsrcbits,
                 table, src_hbm, dst_hbm, zeros_hbm, out_hbm,
                 srcw, dstw, packb,
                 srcidx0, dstidx0, rows0, srcidx1, dstidx1, rows1,
                 acc, sem0, sem1):
    c = lax.axis_index("c")
    s = lax.axis_index("s")
    ept = ne // NS
    accr = acc.shape[0]
    zrows = accr // NS
    orows = rng // NS
    srcmask = (1 << nsrcbits) - 1
    lane = lax.iota(I32, 16)
    pad_vec = lane | ((rng + lane) << nsrcbits)
    trash = packb.shape[0] - 1
    bufs = ((srcidx0, dstidx0, rows0, sem0), (srcidx1, dstidx1, rows1, sem1))

    def start(koff, b):
        srcidx, dstidx, rows, sem = bufs[b]

        def unpack_step(k, _):
            v = packb[pl.ds(koff * chunk + k * 16, 16)]
            srcidx[pl.ds(k * 16, 16)] = v & srcmask
            dstidx[pl.ds(k * 16, 16)] = lax.shift_right_logical(v, nsrcbits)
            return 0
        lax.fori_loop(0, chunk // 16, unpack_step, 0)
        pltpu.async_copy(table.at[srcidx], rows, sem)

    def finish(b):
        srcidx, dstidx, rows, sem = bufs[b]
        pltpu.make_async_copy(table.at[srcidx], rows, sem).wait()
        pltpu.sync_copy(rows, acc.at[dstidx], add=True)

    def flush_many(nch, base):
        @pl.when(nch > 0)
        def _():
            start(base, 0)

            def pipe(k, _):
                @pl.when(k % 2 == 0)
                def _():
                    start(base + k + 1, 1)
                    finish(0)

                @pl.when(k % 2 == 1)
                def _():
                    start(base + k + 1, 0)
                    finish(1)
                return 0
            lax.fori_loop(0, nch - 1, pipe, 0)
            @pl.when((nch - 1) % 2 == 0)
            def _():
                finish(0)
            @pl.when((nch - 1) % 2 == 1)
            def _():
                finish(1)

    for r in range(rounds):
        seg_lo = (r * NC + c) * rng

        def zero_step(j, _):
            pltpu.sync_copy(zeros_hbm, acc.at[pl.ds(s * zrows + j * 64, 64)])
            return 0
        lax.fori_loop(0, zrows // 64, zero_step, 0)
        if zrows % 64:
            pltpu.sync_copy(
                zeros_hbm.at[pl.ds(0, zrows % 64)],
                acc.at[pl.ds(s * zrows + (zrows // 64) * 64, zrows % 64)])
        plsc.subcore_barrier()

        def window_step(w, m):
            pltpu.sync_copy(src_hbm.at[pl.ds(s * ept + w * _W, _W)], srcw)
            pltpu.sync_copy(dst_hbm.at[pl.ds(s * ept + w * _W, _W)], dstw)

            def filt_step(i, m):
                vd = dstw[pl.ds(i * 16, 16)]
                vs = srcw[pl.ds(i * 16, 16)]
                msk = (vd >= seg_lo) & (vd < seg_lo + rng)
                packed = vs | ((vd - seg_lo) << nsrcbits)
                rank = plsc.cumsum(jnp.where(msk, 1, 0))
                idx = jnp.where(msk, m + rank - 1, trash)
                plsc.store_scatter(packb, [idx], packed)
                return m + rank[15]
            m = lax.fori_loop(0, _W // 16, filt_step, m)

            nfull = m // chunk
            flush_many(nfull, 0)
            tail = m - nfull * chunk

            def move_step(k, _):
                packb[pl.ds(k * 16, 16)] = packb[pl.ds(nfull * chunk + k * 16, 16)]
                return 0
            lax.fori_loop(0, jnp.where(nfull > 0, (tail + 15) // 16, 0),
                          move_step, 0)
            return tail
        m = lax.fori_loop(0, ept // _W, window_step, jnp.int32(0))

        mr = ((m + chunk - 1) // chunk) * chunk

        def pad_step(j, _):
            packb[pl.ds(m + j * 16, 16)] = pad_vec
            return 0
        lax.fori_loop(0, (mr - m + 15) // 16, pad_step, 0)
        flush_many(mr // chunk, 0)
        plsc.subcore_barrier()

        pltpu.sync_copy(acc.at[pl.ds(s * orows, orows)],
                        out_hbm.at[pl.ds(seg_lo + s * orows, orows)])
        plsc.subcore_barrier()


def _segsum(table, src, dst, rng, rounds, chunk):
    ne = src.shape[0]
    nseg = rng * NC * rounds
    nsrcbits = 18
    accr = rng + 128
    assert rng % 128 == 0 and chunk % 16 == 0 and (ne // NS) % _W == 0
    zeros = jnp.zeros((64, D), F32)
    mesh = plsc.VectorSubcoreMesh(core_axis_name="c", subcore_axis_name="s")
    body = functools.partial(_segsum_body, ne, rng, rounds, chunk, nsrcbits)
    k = pl.kernel(
        body,
        out_type=jax.ShapeDtypeStruct((nseg, D), F32),
        mesh=mesh,
        scratch_types=[
            pltpu.VMEM((_W,), I32),
            pltpu.VMEM((_W,), I32),
            pltpu.VMEM((_W + chunk + 32,), I32),
            pltpu.VMEM((chunk,), I32),
            pltpu.VMEM((chunk,), I32),
            pltpu.VMEM((chunk, D), F32),
            pltpu.VMEM((chunk,), I32),
            pltpu.VMEM((chunk,), I32),
            pltpu.VMEM((chunk, D), F32),
            pltpu.VMEM_SHARED((accr, D), F32),
            pltpu.SemaphoreType.DMA,
            pltpu.SemaphoreType.DMA,
        ],
        compiler_params=pltpu.CompilerParams(needs_layout_passes=False),
    )
    return k(table, src, dst, zeros)


def _gather_body(ni, chunk, table, idx_hbm, out_hbm, idxv, rows, sem):
    c = lax.axis_index("c")
    s = lax.axis_index("s")
    wid = s * NC + c
    ipt = ni // (NC * NS)
    pltpu.sync_copy(idx_hbm.at[pl.ds(wid * ipt, ipt)], idxv)

    def chunk_step(j, _):
        pltpu.async_copy(table.at[idxv.at[pl.ds(j * chunk, chunk)]],
                         rows, sem).wait()
        pltpu.sync_copy(rows, out_hbm.at[pl.ds(wid * ipt + j * chunk, chunk)])
        return 0
    lax.fori_loop(0, ipt // chunk, chunk_step, 0)


def _gather(table, idx, chunk):
    ni = idx.shape[0]
    mesh = plsc.VectorSubcoreMesh(core_axis_name="c", subcore_axis_name="s")
    body = functools.partial(_gather_body, ni, chunk)
    k = pl.kernel(
        body,
        out_type=jax.ShapeDtypeStruct((ni, D), F32),
        mesh=mesh,
        scratch_types=[
            pltpu.VMEM((ni // (NC * NS),), I32),
            pltpu.VMEM((chunk, D), F32),
            pltpu.SemaphoreType.DMA,
        ],
        compiler_params=pltpu.CompilerParams(needs_layout_passes=False),
    )
    return k(table, idx)


def kernel(x, y, deg_g, deg_lg, edge_index_g, edge_index_lg, pm_pd, Wt, bt, Wg, bg):
    n_nodes, _ = x.shape
    n_edges, _ = y.shape

    wx = jnp.concatenate([Wt[0], Wt[1], Wt[3], Wt[4], Wg[2]], axis=0)
    wy = jnp.concatenate([Wg[0], Wg[1], Wg[3], Wg[4], Wt[2]], axis=0)
    bx = jnp.sum(bt, axis=0).reshape(1, D)
    by = jnp.sum(bg, axis=0).reshape(1, D)
    src_g, dst_g = edge_index_g[0], edge_index_g[1]
    src_l, dst_l = edge_index_lg[0], edge_index_lg[1]
    eids = jnp.arange(n_edges, dtype=I32)

    x0, x1, x2, x3, x4 = _mm5(x, wx, deg_g, 1000)
    y0, y1, y2, y3, y4 = _mm5(y, wy, deg_lg, 1000)

    a1 = _segsum(x3, src_g, dst_g, 5120, 1, 128)[:n_nodes]
    x4t, _ = lax.optimization_barrier((x4, a1))
    px = _gather(x4t, pm_pd.astype(I32), 200)
    y4t, _ = lax.optimization_barrier((y4, px))
    py = _segsum(y4t, eids, dst_g, 5120, 1, 128)[:n_nodes]
    y3t, _ = lax.optimization_barrier((y3, py))
    b1 = _segsum(y3t, src_l, dst_l, 10624, 8, 128)[:n_edges]

    tg = _add(x2, a1, 1000)
    tl = _add(y2, b1, 1000)

    tgt, _ = lax.optimization_barrier((tg, b1))
    a2 = _segsum(tgt, src_g, dst_g, 5120, 1, 128)[:n_nodes]
    tlt, _ = lax.optimization_barrier((tl, a2))
    b2 = _segsum(tlt, src_l, dst_l, 10624, 8, 128)[:n_edges]

    x_new = _assemble(x0, x1, a2, py, bx, 1000)
    y_new = _assemble(y0, y1, b2, px, by, 1000)
    return (x_new, y_new)


# speedup vs baseline: 1.8590x; 1.0676x over previous
import functools

import jax
import jax.numpy as jnp
from jax import lax
from jax.experimental import pallas as pl
from jax.experimental.pallas import tpu as pltpu
from jax.experimental.pallas import tpu_sc as plsc

D = 128
NC = 2
NS = 16
F32 = jnp.float32
I32 = jnp.int32


def _mm5_body(x_ref, w_ref, deg_ref, o0, o1, o2, o3, o4):
    x = x_ref[...]
    p = lax.dot_general(x, w_ref[...], (((1,), (1,)), ((), ())),
                        preferred_element_type=F32)
    o0[...] = p[:, 0 * D:1 * D]
    o1[...] = p[:, 1 * D:2 * D] * deg_ref[...]
    o2[...] = p[:, 2 * D:3 * D]
    o3[...] = p[:, 3 * D:4 * D]
    o4[...] = p[:, 4 * D:5 * D]


def _mm5(x, w, deg, bm):
    m = x.shape[0]
    blk = lambda i: (i, 0)
    out = jax.ShapeDtypeStruct((m, D), F32)
    return pl.pallas_call(
        _mm5_body,
        grid=(m // bm,),
        in_specs=[
            pl.BlockSpec((bm, D), blk),
            pl.BlockSpec((5 * D, D), lambda i: (0, 0)),
            pl.BlockSpec((bm, 1), blk),
        ],
        out_specs=[pl.BlockSpec((bm, D), blk)] * 5,
        out_shape=[out] * 5,
    )(x, w, deg)


def _add_body(a_ref, b_ref, o_ref):
    o_ref[...] = a_ref[...] + b_ref[...]


def _add(a, b, bm):
    m = a.shape[0]
    blk = lambda i: (i, 0)
    return pl.pallas_call(
        _add_body,
        grid=(m // bm,),
        in_specs=[pl.BlockSpec((bm, D), blk)] * 2,
        out_specs=pl.BlockSpec((bm, D), blk),
        out_shape=jax.ShapeDtypeStruct((m, D), F32),
    )(a, b)


def _asm_body(p0_ref, p1_ref, agg_ref, pm_ref, b_ref, o_ref):
    pre = p0_ref[...] + p1_ref[...] + agg_ref[...] + pm_ref[...] + b_ref[...]
    col = lax.broadcasted_iota(I32, pre.shape, 1)
    o_ref[...] = jnp.where((col >= D // 2) & (pre < 0.0), 0.0, pre)


def _assemble(p0, p1, agg, pm, bias, bm):
    m = p0.shape[0]
    blk = lambda i: (i, 0)
    return pl.pallas_call(
        _asm_body,
        grid=(m // bm,),
        in_specs=[pl.BlockSpec((bm, D), blk)] * 4 +
                 [pl.BlockSpec((1, D), lambda i: (0, 0))],
        out_specs=pl.BlockSpec((bm, D), blk),
        out_shape=jax.ShapeDtypeStruct((m, D), F32),
    )(p0, p1, agg, pm, bias)


_W = 2000


def _segsum_body(ne, rng, rounds, chunk, nsrcbits,
                 table, src_hbm, dst_hbm, zeros_hbm, out_hbm,
                 srcw, dstw, packb,
                 srcidx0, dstidx0, rows0, srcidx1, dstidx1, rows1,
                 acc, sem0, sem1):
    c = lax.axis_index("c")
    s = lax.axis_index("s")
    ept = ne // NS
    accr = acc.shape[0]
    zrows = accr // NS
    orows = rng // NS
    srcmask = (1 << nsrcbits) - 1
    lane = lax.iota(I32, 16)
    pad_vec = lane | ((rng + lane) << nsrcbits)
    trash = packb.shape[0] - 1
    bufs = ((srcidx0, dstidx0, rows0, sem0), (srcidx1, dstidx1, rows1, sem1))

    def start(koff, b):
        srcidx, dstidx, rows, sem = bufs[b]

        def unpack_step(k, _):
            v = packb[pl.ds(koff * chunk + k * 16, 16)]
            srcidx[pl.ds(k * 16, 16)] = v & srcmask
            dstidx[pl.ds(k * 16, 16)] = lax.shift_right_logical(v, nsrcbits)
            return 0
        lax.fori_loop(0, chunk // 16, unpack_step, 0)
        pltpu.async_copy(table.at[srcidx], rows, sem)

    def finish(b):
        srcidx, dstidx, rows, sem = bufs[b]
        pltpu.make_async_copy(table.at[srcidx], rows, sem).wait()
        pltpu.sync_copy(rows, acc.at[dstidx], add=True)

    def flush_many(nch, base):
        @pl.when(nch > 0)
        def _():
            start(base, 0)

            def pipe(k, _):
                @pl.when(k % 2 == 0)
                def _():
                    start(base + k + 1, 1)
                    finish(0)

                @pl.when(k % 2 == 1)
                def _():
                    start(base + k + 1, 0)
                    finish(1)
                return 0
            lax.fori_loop(0, nch - 1, pipe, 0)
            @pl.when((nch - 1) % 2 == 0)
            def _():
                finish(0)
            @pl.when((nch - 1) % 2 == 1)
            def _():
                finish(1)

    for r in range(rounds):
        seg_lo = (r * NC + c) * rng

        def zero_step(j, _):
            pltpu.sync_copy(zeros_hbm, acc.at[pl.ds(s * zrows + j * 64, 64)])
            return 0
        lax.fori_loop(0, zrows // 64, zero_step, 0)
        if zrows % 64:
            pltpu.sync_copy(
                zeros_hbm.at[pl.ds(0, zrows % 64)],
                acc.at[pl.ds(s * zrows + (zrows // 64) * 64, zrows % 64)])
        plsc.subcore_barrier()

        def window_step(w, m):
            pltpu.sync_copy(src_hbm.at[pl.ds(s * ept + w * _W, _W)], srcw)
            pltpu.sync_copy(dst_hbm.at[pl.ds(s * ept + w * _W, _W)], dstw)

            def filt_step(i, m):
                vd = dstw[pl.ds(i * 16, 16)]
                vs = srcw[pl.ds(i * 16, 16)]
                msk = (vd >= seg_lo) & (vd < seg_lo + rng)
                packed = vs | ((vd - seg_lo) << nsrcbits)
                rank = plsc.cumsum(jnp.where(msk, 1, 0))
                idx = jnp.where(msk, m + rank - 1, trash)
                plsc.store_scatter(packb, [idx], packed)
                return m + rank[15]
            m = lax.fori_loop(0, _W // 16, filt_step, m)

            nfull = m // chunk
            flush_many(nfull, 0)
            tail = m - nfull * chunk

            def move_step(k, _):
                packb[pl.ds(k * 16, 16)] = packb[pl.ds(nfull * chunk + k * 16, 16)]
                return 0
            lax.fori_loop(0, jnp.where(nfull > 0, (tail + 15) // 16, 0),
                          move_step, 0)
            return tail
        m = lax.fori_loop(0, ept // _W, window_step, jnp.int32(0))

        mr = ((m + chunk - 1) // chunk) * chunk

        def pad_step(j, _):
            packb[pl.ds(m + j * 16, 16)] = pad_vec
            return 0
        lax.fori_loop(0, (mr - m + 15) // 16, pad_step, 0)
        flush_many(mr // chunk, 0)
        plsc.subcore_barrier()

        pltpu.sync_copy(acc.at[pl.ds(s * orows, orows)],
                        out_hbm.at[pl.ds(seg_lo + s * orows, orows)])
        plsc.subcore_barrier()


def _segsum(table, src, dst, rng, rounds, chunk):
    ne = src.shape[0]
    nseg = rng * NC * rounds
    nsrcbits = 18
    accr = rng + 128
    assert rng % 128 == 0 and chunk % 16 == 0 and (ne // NS) % _W == 0
    zeros = jnp.zeros((64, D), F32)
    mesh = plsc.VectorSubcoreMesh(core_axis_name="c", subcore_axis_name="s")
    body = functools.partial(_segsum_body, ne, rng, rounds, chunk, nsrcbits)
    k = pl.kernel(
        body,
        out_type=jax.ShapeDtypeStruct((nseg, D), F32),
        mesh=mesh,
        scratch_types=[
            pltpu.VMEM((_W,), I32),
            pltpu.VMEM((_W,), I32),
            pltpu.VMEM((_W + chunk + 32,), I32),
            pltpu.VMEM((chunk,), I32),
            pltpu.VMEM((chunk,), I32),
            pltpu.VMEM((chunk, D), F32),
            pltpu.VMEM((chunk,), I32),
            pltpu.VMEM((chunk,), I32),
            pltpu.VMEM((chunk, D), F32),
            pltpu.VMEM_SHARED((accr, D), F32),
            pltpu.SemaphoreType.DMA,
            pltpu.SemaphoreType.DMA,
        ],
        compiler_params=pltpu.CompilerParams(needs_layout_passes=False),
    )
    return k(table, src, dst, zeros)


def _segsum2_body(ne, rng, nsrcbits, chunk,
                  t1, t2, src_hbm, dst_hbm, zeros_hbm, out1_hbm, out2_hbm,
                  srcw, dstw, packb, packe,
                  srcidx, dstidx, eidx, rows1, rows2, acc1, acc2, sem1, sem2):
    c = lax.axis_index("c")
    s = lax.axis_index("s")
    ept = ne // NS
    accr = acc1.shape[0]
    zrows = accr // NS
    orows = rng // NS
    srcmask = (1 << nsrcbits) - 1
    lane = lax.iota(I32, 16)
    pad_vec = lane | ((rng + lane) << nsrcbits)
    trash = packb.shape[0] - 1
    seg_lo = c * rng

    def flush(koff, _):
        def unpack_step(k, _):
            v = packb[pl.ds(koff * chunk + k * 16, 16)]
            srcidx[pl.ds(k * 16, 16)] = v & srcmask
            dstidx[pl.ds(k * 16, 16)] = lax.shift_right_logical(v, nsrcbits)
            eidx[pl.ds(k * 16, 16)] = packe[pl.ds(koff * chunk + k * 16, 16)]
            return 0
        lax.fori_loop(0, chunk // 16, unpack_step, 0)
        pltpu.async_copy(t1.at[srcidx], rows1, sem1)
        pltpu.async_copy(t2.at[eidx], rows2, sem2)
        pltpu.make_async_copy(t1.at[srcidx], rows1, sem1).wait()
        pltpu.sync_copy(rows1, acc1.at[dstidx], add=True)
        pltpu.make_async_copy(t2.at[eidx], rows2, sem2).wait()
        pltpu.sync_copy(rows2, acc2.at[dstidx], add=True)
        return 0

    def zero_step(j, _):
        pltpu.sync_copy(zeros_hbm, acc1.at[pl.ds(s * zrows + j * 64, 64)])
        pltpu.sync_copy(zeros_hbm, acc2.at[pl.ds(s * zrows + j * 64, 64)])
        return 0
    lax.fori_loop(0, zrows // 64, zero_step, 0)
    if zrows % 64:
        pltpu.sync_copy(zeros_hbm.at[pl.ds(0, zrows % 64)],
                        acc1.at[pl.ds(s * zrows + (zrows // 64) * 64, zrows % 64)])
        pltpu.sync_copy(zeros_hbm.at[pl.ds(0, zrows % 64)],
                        acc2.at[pl.ds(s * zrows + (zrows // 64) * 64, zrows % 64)])
    plsc.subcore_barrier()

    def window_step(w, m):
        pltpu.sync_copy(src_hbm.at[pl.ds(s * ept + w * _W, _W)], srcw)
        pltpu.sync_copy(dst_hbm.at[pl.ds(s * ept + w * _W, _W)], dstw)

        def filt_step(i, m):
            vd = dstw[pl.ds(i * 16, 16)]
            vs = srcw[pl.ds(i * 16, 16)]
            ve = (s * ept + w * _W + i * 16) + lane
            msk = (vd >= seg_lo) & (vd < seg_lo + rng)
            packed = vs | ((vd - seg_lo) << nsrcbits)
            rank = plsc.cumsum(jnp.where(msk, 1, 0))
            idx = jnp.where(msk, m + rank - 1, trash)
            plsc.store_scatter(packb, [idx], packed)
            plsc.store_scatter(packe, [idx], ve)
            return m + rank[15]
        m = lax.fori_loop(0, _W // 16, filt_step, m)

        nfull = m // chunk
        lax.fori_loop(0, nfull, flush, 0)
        tail = m - nfull * chunk

        def move_step(k, _):
            packb[pl.ds(k * 16, 16)] = packb[pl.ds(nfull * chunk + k * 16, 16)]
            packe[pl.ds(k * 16, 16)] = packe[pl.ds(nfull * chunk + k * 16, 16)]
            return 0
        lax.fori_loop(0, jnp.where(nfull > 0, (tail + 15) // 16, 0), move_step, 0)
        return tail
    m = lax.fori_loop(0, ept // _W, window_step, jnp.int32(0))

    mr = ((m + chunk - 1) // chunk) * chunk

    def pad_step(j, _):
        packb[pl.ds(m + j * 16, 16)] = pad_vec
        packe[pl.ds(m + j * 16, 16)] = lane
        return 0
    lax.fori_loop(0, (mr - m + 15) // 16, pad_step, 0)
    lax.fori_loop(0, mr // chunk, flush, 0)
    plsc.subcore_barrier()

    pltpu.sync_copy(acc1.at[pl.ds(s * orows, orows)],
                    out1_hbm.at[pl.ds(seg_lo + s * orows, orows)])
    pltpu.sync_copy(acc2.at[pl.ds(s * orows, orows)],
                    out2_hbm.at[pl.ds(seg_lo + s * orows, orows)])
    plsc.subcore_barrier()


def _segsum2(t1, t2, src, dst, rng, chunk):
    ne = src.shape[0]
    nseg = rng * NC
    nsrcbits = 18
    accr = rng + 128
    assert rng % 128 == 0 and chunk % 16 == 0 and (ne // NS) % _W == 0
    zeros = jnp.zeros((64, D), F32)
    mesh = plsc.VectorSubcoreMesh(core_axis_name="c", subcore_axis_name="s")
    body = functools.partial(_segsum2_body, ne, rng, nsrcbits, chunk)
    k = pl.kernel(
        body,
        out_type=(jax.ShapeDtypeStruct((nseg, D), F32),
                  jax.ShapeDtypeStruct((nseg, D), F32)),
        mesh=mesh,
        scratch_types=[
            pltpu.VMEM((_W,), I32),
            pltpu.VMEM((_W,), I32),
            pltpu.VMEM((_W + chunk + 32,), I32),
            pltpu.VMEM((_W + chunk + 32,), I32),
            pltpu.VMEM((chunk,), I32),
            pltpu.VMEM((chunk,), I32),
            pltpu.VMEM((chunk,), I32),
            pltpu.VMEM((chunk, D), F32),
            pltpu.VMEM((chunk, D), F32),
            pltpu.VMEM_SHARED((accr, D), F32),
            pltpu.VMEM_SHARED((accr, D), F32),
            pltpu.SemaphoreType.DMA,
            pltpu.SemaphoreType.DMA,
        ],
        compiler_params=pltpu.CompilerParams(needs_layout_passes=False),
    )
    return k(t1, t2, src, dst, zeros)


def _gather_body(ni, chunk, table, idx_hbm, out_hbm, idxv, rows, sem):
    c = lax.axis_index("c")
    s = lax.axis_index("s")
    wid = s * NC + c
    ipt = ni // (NC * NS)
    pltpu.sync_copy(idx_hbm.at[pl.ds(wid * ipt, ipt)], idxv)

    def chunk_step(j, _):
        pltpu.async_copy(table.at[idxv.at[pl.ds(j * chunk, chunk)]],
                         rows, sem).wait()
        pltpu.sync_copy(rows, out_hbm.at[pl.ds(wid * ipt + j * chunk, chunk)])
        return 0
    lax.fori_loop(0, ipt // chunk, chunk_step, 0)


def _gather(table, idx, chunk):
    ni = idx.shape[0]
    mesh = plsc.VectorSubcoreMesh(core_axis_name="c", subcore_axis_name="s")
    body = functools.partial(_gather_body, ni, chunk)
    k = pl.kernel(
        body,
        out_type=jax.ShapeDtypeStruct((ni, D), F32),
        mesh=mesh,
        scratch_types=[
            pltpu.VMEM((ni // (NC * NS),), I32),
            pltpu.VMEM((chunk, D), F32),
            pltpu.SemaphoreType.DMA,
        ],
        compiler_params=pltpu.CompilerParams(needs_layout_passes=False),
    )
    return k(table, idx)


def kernel(x, y, deg_g, deg_lg, edge_index_g, edge_index_lg, pm_pd, Wt, bt, Wg, bg):
    n_nodes, _ = x.shape
    n_edges, _ = y.shape

    wx = jnp.concatenate([Wt[0], Wt[1], Wt[3], Wt[4], Wg[2]], axis=0)
    wy = jnp.concatenate([Wg[0], Wg[1], Wg[3], Wg[4], Wt[2]], axis=0)
    bx = jnp.sum(bt, axis=0).reshape(1, D)
    by = jnp.sum(bg, axis=0).reshape(1, D)
    src_g, dst_g = edge_index_g[0], edge_index_g[1]
    src_l, dst_l = edge_index_lg[0], edge_index_lg[1]
    x0, x1, x2, x3, x4 = _mm5(x, wx, deg_g, 1000)
    y0, y1, y2, y3, y4 = _mm5(y, wy, deg_lg, 1000)

    a1p, pyp = _segsum2(x3, y4, src_g, dst_g, 5120, 128)
    a1, py = a1p[:n_nodes], pyp[:n_nodes]
    x4t, _ = lax.optimization_barrier((x4, a1))
    px = _gather(x4t, pm_pd.astype(I32), 200)
    y3t, _ = lax.optimization_barrier((y3, px))
    b1 = _segsum(y3t, src_l, dst_l, 10624, 8, 128)[:n_edges]

    tg = _add(x2, a1, 1000)
    tl = _add(y2, b1, 1000)

    tgt, _ = lax.optimization_barrier((tg, b1))
    a2 = _segsum(tgt, src_g, dst_g, 5120, 1, 128)[:n_nodes]
    tlt, _ = lax.optimization_barrier((tl, a2))
    b2 = _segsum(tlt, src_l, dst_l, 10624, 8, 128)[:n_edges]

    x_new = _assemble(x0, x1, a2, py, bx, 1000)
    y_new = _assemble(y0, y1, b2, px, by, 1000)
    return (x_new, y_new)


# speedup vs baseline: 2.0181x; 1.0856x over previous
import functools

import jax
import jax.numpy as jnp
from jax import lax
from jax.experimental import pallas as pl
from jax.experimental.pallas import tpu as pltpu
from jax.experimental.pallas import tpu_sc as plsc

D = 128
NC = 2
NS = 16
F32 = jnp.float32
I32 = jnp.int32


def _mm5_body(x_ref, w_ref, deg_ref, o0, o1, o2, o3, o4):
    x = x_ref[...]
    p = lax.dot_general(x, w_ref[...], (((1,), (1,)), ((), ())),
                        preferred_element_type=F32)
    o0[...] = p[:, 0 * D:1 * D]
    o1[...] = p[:, 1 * D:2 * D] * deg_ref[...]
    o2[...] = p[:, 2 * D:3 * D]
    o3[...] = p[:, 3 * D:4 * D]
    o4[...] = p[:, 4 * D:5 * D]


def _mm5(x, w, deg, bm):
    m = x.shape[0]
    blk = lambda i: (i, 0)
    out = jax.ShapeDtypeStruct((m, D), F32)
    return pl.pallas_call(
        _mm5_body,
        grid=(m // bm,),
        in_specs=[
            pl.BlockSpec((bm, D), blk),
            pl.BlockSpec((5 * D, D), lambda i: (0, 0)),
            pl.BlockSpec((bm, 1), blk),
        ],
        out_specs=[pl.BlockSpec((bm, D), blk)] * 5,
        out_shape=[out] * 5,
    )(x, w, deg)


def _add_body(a_ref, b_ref, o_ref):
    o_ref[...] = a_ref[...] + b_ref[...]


def _add(a, b, bm):
    m = a.shape[0]
    blk = lambda i: (i, 0)
    return pl.pallas_call(
        _add_body,
        grid=(m // bm,),
        in_specs=[pl.BlockSpec((bm, D), blk)] * 2,
        out_specs=pl.BlockSpec((bm, D), blk),
        out_shape=jax.ShapeDtypeStruct((m, D), F32),
    )(a, b)


def _asm_body(p0_ref, p1_ref, agg_ref, pm_ref, b_ref, o_ref):
    pre = p0_ref[...] + p1_ref[...] + agg_ref[...] + pm_ref[...] + b_ref[...]
    col = lax.broadcasted_iota(I32, pre.shape, 1)
    o_ref[...] = jnp.where((col >= D // 2) & (pre < 0.0), 0.0, pre)


def _assemble(p0, p1, agg, pm, bias, bm):
    m = p0.shape[0]
    blk = lambda i: (i, 0)
    return pl.pallas_call(
        _asm_body,
        grid=(m // bm,),
        in_specs=[pl.BlockSpec((bm, D), blk)] * 4 +
                 [pl.BlockSpec((1, D), lambda i: (0, 0))],
        out_specs=pl.BlockSpec((bm, D), blk),
        out_shape=jax.ShapeDtypeStruct((m, D), F32),
    )(p0, p1, agg, pm, bias)


_W = 2000


def _segsum_body(ne, rng, rounds, chunk, nsrcbits,
                 table, src_hbm, dst_hbm, zeros_hbm, out_hbm,
                 srcw, dstw, packb,
                 srcidx0, dstidx0, rows0, srcidx1, dstidx1, rows1,
                 zbuf, acc, sem0, sem1):
    c = lax.axis_index("c")
    s = lax.axis_index("s")
    ept = ne // NS
    accr = acc.shape[0]
    zrows = accr // NS
    orows = rng // NS
    srcmask = (1 << nsrcbits) - 1
    lane = lax.iota(I32, 16)
    pad_vec = lane | ((rng + lane) << nsrcbits)
    trash = packb.shape[0] - 1
    bufs = ((srcidx0, dstidx0, rows0, sem0), (srcidx1, dstidx1, rows1, sem1))

    def start(koff, b):
        srcidx, dstidx, rows, sem = bufs[b]

        def unpack_step(k, _):
            v = packb[pl.ds(koff * chunk + k * 16, 16)]
            srcidx[pl.ds(k * 16, 16)] = v & srcmask
            dstidx[pl.ds(k * 16, 16)] = lax.shift_right_logical(v, nsrcbits)
            return 0
        lax.fori_loop(0, chunk // 16, unpack_step, 0)
        pltpu.async_copy(table.at[srcidx], rows, sem)

    def finish(b):
        srcidx, dstidx, rows, sem = bufs[b]
        pltpu.make_async_copy(table.at[srcidx], rows, sem).wait()
        pltpu.sync_copy(rows, acc.at[dstidx], add=True)

    def flush_many(nch, base):
        @pl.when(nch > 0)
        def _():
            start(base, 0)

            def pipe(k, _):
                @pl.when(k % 2 == 0)
                def _():
                    start(base + k + 1, 1)
                    finish(0)

                @pl.when(k % 2 == 1)
                def _():
                    start(base + k + 1, 0)
                    finish(1)
                return 0
            lax.fori_loop(0, nch - 1, pipe, 0)
            @pl.when((nch - 1) % 2 == 0)
            def _():
                finish(0)
            @pl.when((nch - 1) % 2 == 1)
            def _():
                finish(1)

    pltpu.sync_copy(zeros_hbm.at[pl.ds(0, 32)], zbuf)

    for r in range(rounds):
        seg_lo = (r * NC + c) * rng

        def zero_step(j, _):
            pltpu.sync_copy(zbuf, acc.at[pl.ds(s * zrows + j * 32, 32)])
            return 0
        lax.fori_loop(0, zrows // 32, zero_step, 0)
        if zrows % 32:
            pltpu.sync_copy(
                zbuf.at[pl.ds(0, zrows % 32)],
                acc.at[pl.ds(s * zrows + (zrows // 32) * 32, zrows % 32)])
        plsc.subcore_barrier()

        def window_step(w, m):
            pltpu.sync_copy(src_hbm.at[pl.ds(s * ept + w * _W, _W)], srcw)
            pltpu.sync_copy(dst_hbm.at[pl.ds(s * ept + w * _W, _W)], dstw)

            def filt_step(i, m):
                vd = dstw[pl.ds(i * 16, 16)]
                vs = srcw[pl.ds(i * 16, 16)]
                msk = (vd >= seg_lo) & (vd < seg_lo + rng)
                packed = vs | ((vd - seg_lo) << nsrcbits)
                rank = plsc.cumsum(jnp.where(msk, 1, 0))
                idx = jnp.where(msk, m + rank - 1, trash)
                plsc.store_scatter(packb, [idx], packed)
                return m + rank[15]
            m = lax.fori_loop(0, _W // 16, filt_step, m)

            nfull = m // chunk
            flush_many(nfull, 0)
            tail = m - nfull * chunk

            def move_step(k, _):
                packb[pl.ds(k * 16, 16)] = packb[pl.ds(nfull * chunk + k * 16, 16)]
                return 0
            lax.fori_loop(0, jnp.where(nfull > 0, (tail + 15) // 16, 0),
                          move_step, 0)
            return tail
        m = lax.fori_loop(0, ept // _W, window_step, jnp.int32(0))

        mr = ((m + chunk - 1) // chunk) * chunk

        def pad_step(j, _):
            packb[pl.ds(m + j * 16, 16)] = pad_vec
            return 0
        lax.fori_loop(0, (mr - m + 15) // 16, pad_step, 0)
        flush_many(mr // chunk, 0)
        plsc.subcore_barrier()

        pltpu.sync_copy(acc.at[pl.ds(s * orows, orows)],
                        out_hbm.at[pl.ds(seg_lo + s * orows, orows)])
        plsc.subcore_barrier()


def _segsum(table, src, dst, rng, rounds, chunk):
    ne = src.shape[0]
    nseg = rng * NC * rounds
    nsrcbits = 18
    accr = rng + 128
    assert rng % 128 == 0 and chunk % 16 == 0 and (ne // NS) % _W == 0
    zeros = jnp.zeros((64, D), F32)
    mesh = plsc.VectorSubcoreMesh(core_axis_name="c", subcore_axis_name="s")
    body = functools.partial(_segsum_body, ne, rng, rounds, chunk, nsrcbits)
    k = pl.kernel(
        body,
        out_type=jax.ShapeDtypeStruct((nseg, D), F32),
        mesh=mesh,
        scratch_types=[
            pltpu.VMEM((_W,), I32),
            pltpu.VMEM((_W,), I32),
            pltpu.VMEM((_W + chunk + 32,), I32),
            pltpu.VMEM((chunk,), I32),
            pltpu.VMEM((chunk,), I32),
            pltpu.VMEM((chunk, D), F32),
            pltpu.VMEM((chunk,), I32),
            pltpu.VMEM((chunk,), I32),
            pltpu.VMEM((chunk, D), F32),
            pltpu.VMEM((32, D), F32),
            pltpu.VMEM_SHARED((accr, D), F32),
            pltpu.SemaphoreType.DMA,
            pltpu.SemaphoreType.DMA,
        ],
        compiler_params=pltpu.CompilerParams(needs_layout_passes=False),
    )
    return k(table, src, dst, zeros)


def _segsum2_body(ne, rng, nsrcbits, chunk,
                  t1, t2, src_hbm, dst_hbm, zeros_hbm, out1_hbm, out2_hbm,
                  srcw, dstw, packb, packe,
                  srcidx, dstidx, eidx, rows1, rows2, zbuf,
                  acc1, acc2, sem1, sem2):
    c = lax.axis_index("c")
    s = lax.axis_index("s")
    ept = ne // NS
    accr = acc1.shape[0]
    zrows = accr // NS
    orows = rng // NS
    srcmask = (1 << nsrcbits) - 1
    lane = lax.iota(I32, 16)
    pad_vec = lane | ((rng + lane) << nsrcbits)
    trash = packb.shape[0] - 1
    seg_lo = c * rng

    def flush(koff, _):
        def unpack_step(k, _):
            v = packb[pl.ds(koff * chunk + k * 16, 16)]
            srcidx[pl.ds(k * 16, 16)] = v & srcmask
            dstidx[pl.ds(k * 16, 16)] = lax.shift_right_logical(v, nsrcbits)
            eidx[pl.ds(k * 16, 16)] = packe[pl.ds(koff * chunk + k * 16, 16)]
            return 0
        lax.fori_loop(0, chunk // 16, unpack_step, 0)
        pltpu.async_copy(t1.at[srcidx], rows1, sem1)
        pltpu.async_copy(t2.at[eidx], rows2, sem2)
        pltpu.make_async_copy(t1.at[srcidx], rows1, sem1).wait()
        pltpu.sync_copy(rows1, acc1.at[dstidx], add=True)
        pltpu.make_async_copy(t2.at[eidx], rows2, sem2).wait()
        pltpu.sync_copy(rows2, acc2.at[dstidx], add=True)
        return 0

    pltpu.sync_copy(zeros_hbm.at[pl.ds(0, 32)], zbuf)

    def zero_step(j, _):
        pltpu.sync_copy(zbuf, acc1.at[pl.ds(s * zrows + j * 32, 32)])
        pltpu.sync_copy(zbuf, acc2.at[pl.ds(s * zrows + j * 32, 32)])
        return 0
    lax.fori_loop(0, zrows // 32, zero_step, 0)
    if zrows % 32:
        pltpu.sync_copy(zbuf.at[pl.ds(0, zrows % 32)],
                        acc1.at[pl.ds(s * zrows + (zrows // 32) * 32, zrows % 32)])
        pltpu.sync_copy(zbuf.at[pl.ds(0, zrows % 32)],
                        acc2.at[pl.ds(s * zrows + (zrows // 32) * 32, zrows % 32)])
    plsc.subcore_barrier()

    def window_step(w, m):
        pltpu.sync_copy(src_hbm.at[pl.ds(s * ept + w * _W, _W)], srcw)
        pltpu.sync_copy(dst_hbm.at[pl.ds(s * ept + w * _W, _W)], dstw)

        def filt_step(i, m):
            vd = dstw[pl.ds(i * 16, 16)]
            vs = srcw[pl.ds(i * 16, 16)]
            ve = (s * ept + w * _W + i * 16) + lane
            msk = (vd >= seg_lo) & (vd < seg_lo + rng)
            packed = vs | ((vd - seg_lo) << nsrcbits)
            rank = plsc.cumsum(jnp.where(msk, 1, 0))
            idx = jnp.where(msk, m + rank - 1, trash)
            plsc.store_scatter(packb, [idx], packed)
            plsc.store_scatter(packe, [idx], ve)
            return m + rank[15]
        m = lax.fori_loop(0, _W // 16, filt_step, m)

        nfull = m // chunk
        lax.fori_loop(0, nfull, flush, 0)
        tail = m - nfull * chunk

        def move_step(k, _):
            packb[pl.ds(k * 16, 16)] = packb[pl.ds(nfull * chunk + k * 16, 16)]
            packe[pl.ds(k * 16, 16)] = packe[pl.ds(nfull * chunk + k * 16, 16)]
            return 0
        lax.fori_loop(0, jnp.where(nfull > 0, (tail + 15) // 16, 0), move_step, 0)
        return tail
    m = lax.fori_loop(0, ept // _W, window_step, jnp.int32(0))

    mr = ((m + chunk - 1) // chunk) * chunk

    def pad_step(j, _):
        packb[pl.ds(m + j * 16, 16)] = pad_vec
        packe[pl.ds(m + j * 16, 16)] = lane
        return 0
    lax.fori_loop(0, (mr - m + 15) // 16, pad_step, 0)
    lax.fori_loop(0, mr // chunk, flush, 0)
    plsc.subcore_barrier()

    pltpu.sync_copy(acc1.at[pl.ds(s * orows, orows)],
                    out1_hbm.at[pl.ds(seg_lo + s * orows, orows)])
    pltpu.sync_copy(acc2.at[pl.ds(s * orows, orows)],
                    out2_hbm.at[pl.ds(seg_lo + s * orows, orows)])
    plsc.subcore_barrier()


def _segsum2(t1, t2, src, dst, rng, chunk):
    ne = src.shape[0]
    nseg = rng * NC
    nsrcbits = 18
    accr = rng + 128
    assert rng % 128 == 0 and chunk % 16 == 0 and (ne // NS) % _W == 0
    zeros = jnp.zeros((64, D), F32)
    mesh = plsc.VectorSubcoreMesh(core_axis_name="c", subcore_axis_name="s")
    body = functools.partial(_segsum2_body, ne, rng, nsrcbits, chunk)
    k = pl.kernel(
        body,
        out_type=(jax.ShapeDtypeStruct((nseg, D), F32),
                  jax.ShapeDtypeStruct((nseg, D), F32)),
        mesh=mesh,
        scratch_types=[
            pltpu.VMEM((_W,), I32),
            pltpu.VMEM((_W,), I32),
            pltpu.VMEM((_W + chunk + 32,), I32),
            pltpu.VMEM((_W + chunk + 32,), I32),
            pltpu.VMEM((chunk,), I32),
            pltpu.VMEM((chunk,), I32),
            pltpu.VMEM((chunk,), I32),
            pltpu.VMEM((chunk, D), F32),
            pltpu.VMEM((chunk, D), F32),
            pltpu.VMEM((32, D), F32),
            pltpu.VMEM_SHARED((accr, D), F32),
            pltpu.VMEM_SHARED((accr, D), F32),
            pltpu.SemaphoreType.DMA,
            pltpu.SemaphoreType.DMA,
        ],
        compiler_params=pltpu.CompilerParams(needs_layout_passes=False),
    )
    return k(t1, t2, src, dst, zeros)


def _gather_body(ni, chunk, table, idx_hbm, out_hbm, idxv, rows, sem):
    c = lax.axis_index("c")
    s = lax.axis_index("s")
    wid = s * NC + c
    ipt = ni // (NC * NS)
    pltpu.sync_copy(idx_hbm.at[pl.ds(wid * ipt, ipt)], idxv)

    def chunk_step(j, _):
        pltpu.async_copy(table.at[idxv.at[pl.ds(j * chunk, chunk)]],
                         rows, sem).wait()
        pltpu.sync_copy(rows, out_hbm.at[pl.ds(wid * ipt + j * chunk, chunk)])
        return 0
    lax.fori_loop(0, ipt // chunk, chunk_step, 0)


def _gather(table, idx, chunk):
    ni = idx.shape[0]
    mesh = plsc.VectorSubcoreMesh(core_axis_name="c", subcore_axis_name="s")
    body = functools.partial(_gather_body, ni, chunk)
    k = pl.kernel(
        body,
        out_type=jax.ShapeDtypeStruct((ni, D), F32),
        mesh=mesh,
        scratch_types=[
            pltpu.VMEM((ni // (NC * NS),), I32),
            pltpu.VMEM((chunk, D), F32),
            pltpu.SemaphoreType.DMA,
        ],
        compiler_params=pltpu.CompilerParams(needs_layout_passes=False),
    )
    return k(table, idx)


def kernel(x, y, deg_g, deg_lg, edge_index_g, edge_index_lg, pm_pd, Wt, bt, Wg, bg):
    n_nodes, _ = x.shape
    n_edges, _ = y.shape

    wx = jnp.concatenate([Wt[0], Wt[1], Wt[3], Wt[4], Wg[2]], axis=0)
    wy = jnp.concatenate([Wg[0], Wg[1], Wg[3], Wg[4], Wt[2]], axis=0)
    bx = jnp.sum(bt, axis=0).reshape(1, D)
    by = jnp.sum(bg, axis=0).reshape(1, D)
    src_g, dst_g = edge_index_g[0], edge_index_g[1]
    src_l, dst_l = edge_index_lg[0], edge_index_lg[1]
    x0, x1, x2, x3, x4 = _mm5(x, wx, deg_g, 1000)
    y0, y1, y2, y3, y4 = _mm5(y, wy, deg_lg, 1000)

    a1p, pyp = _segsum2(x3, y4, src_g, dst_g, 5120, 128)
    a1, py = a1p[:n_nodes], pyp[:n_nodes]
    x4t, _ = lax.optimization_barrier((x4, a1))
    px = _gather(x4t, pm_pd.astype(I32), 200)
    y3t, _ = lax.optimization_barrier((y3, px))
    b1 = _segsum(y3t, src_l, dst_l, 10624, 8, 128)[:n_edges]

    tg = _add(x2, a1, 1000)
    tl = _add(y2, b1, 1000)

    tgt, _ = lax.optimization_barrier((tg, b1))
    a2 = _segsum(tgt, src_g, dst_g, 5120, 1, 128)[:n_nodes]
    tlt, _ = lax.optimization_barrier((tl, a2))
    b2 = _segsum(tlt, src_l, dst_l, 10624, 8, 128)[:n_edges]

    x_new = _assemble(x0, x1, a2, py, bx, 1000)
    y_new = _assemble(y0, y1, b2, px, by, 1000)
    return (x_new, y_new)
